# baseline (speedup 1.0000x reference)
import jax
import jax.numpy as jnp
from jax.experimental import pallas as pl
from jax.experimental.pallas import tpu as pltpu

_ROWS = 512
_K = 4096
_CHUNK = 2048
_D = 32
_NCB = 2


def _chunk_min(dist, col):
    v = jnp.min(dist, axis=1, keepdims=True)
    i = jnp.min(jnp.where(dist <= v, col, _K), axis=1, keepdims=True)
    return v, i


def _vq_kernel(flat_ref, z2_ref, embedt_ref, embed_ref, e2_ref, out_ref):
    z = flat_ref[0]
    z2 = z2_ref[0]
    et = embedt_ref[0]
    e = embed_ref[0]
    e2 = e2_ref[0]
    s = jax.lax.dot_general(z.astype(jnp.bfloat16), et, (((1,), (0,)), ((), ())),
                            preferred_element_type=jnp.float32)
    dist = (z2 - 2.0 * s) + e2
    col = jax.lax.broadcasted_iota(jnp.int32, (dist.shape[0], _CHUNK), 1)
    v0, i0 = _chunk_min(dist[:, :_CHUNK], col)
    v1, i1 = _chunk_min(dist[:, _CHUNK:], col)
    v0b = v0.astype(jnp.bfloat16).astype(jnp.float32)
    idx = jnp.where(v1 < v0b, i1 + _CHUNK, i0)
    colk = jax.lax.broadcasted_iota(jnp.int32, dist.shape, 1)
    onehot = (colk == idx).astype(jnp.float32)
    out_ref[0] = jax.lax.dot_general(onehot, e, (((1,), (0,)), ((), ())),
                                     preferred_element_type=jnp.float32)


def kernel(z_e, embed0, embed1):
    B, C, H, W = z_e.shape
    d = C // _NCB
    n = B * H * W
    flat = jnp.transpose(z_e.reshape(B, _NCB, d, H, W),
                         (1, 0, 3, 4, 2)).reshape(_NCB, n, d)
    z2 = jnp.sum(flat * flat, axis=2)[:, :, None]
    embeds = jnp.stack([embed0, embed1])
    embeds_t = jnp.transpose(embeds, (0, 2, 1)).astype(jnp.bfloat16)
    e2 = jnp.sum(embeds * embeds, axis=2)[:, None, :]

    quant = pl.pallas_call(
        _vq_kernel,
        grid=(_NCB, n // _ROWS),
        in_specs=[
            pl.BlockSpec((1, _ROWS, d), lambda c, t: (c, t, 0)),
            pl.BlockSpec((1, _ROWS, 1), lambda c, t: (c, t, 0)),
            pl.BlockSpec((1, d, _K), lambda c, t: (c, 0, 0)),
            pl.BlockSpec((1, _K, d), lambda c, t: (c, 0, 0)),
            pl.BlockSpec((1, 1, _K), lambda c, t: (c, 0, 0)),
        ],
        out_specs=pl.BlockSpec((1, _ROWS, d), lambda c, t: (c, t, 0)),
        out_shape=jax.ShapeDtypeStruct((_NCB, n, d), jnp.float32),
    )(flat, z2, embeds_t, embeds, e2)

    z_q = jnp.transpose(quant.reshape(_NCB, B, H, W, d),
                        (1, 0, 4, 2, 3)).reshape(B, C, H, W)
    return z_q


# speedup vs baseline: 1.2806x; 1.2806x over previous
import jax
import jax.numpy as jnp
from jax.experimental import pallas as pl
from jax.experimental.pallas import tpu as pltpu
from jax.experimental.pallas import tpu_sc as plsc

_ROWS = 512
_K = 4096
_CHUNK = 2048
_D = 32
_NCB = 2
_GW = 128


def _chunk_min(dist, col):
    v = jnp.min(dist, axis=1, keepdims=True)
    i = jnp.min(jnp.where(dist <= v, col, _K), axis=1, keepdims=True)
    return v, i


def _argmin_kernel(flat_ref, z2_ref, embedt_ref, e2_ref, out_ref):
    z = flat_ref[...]
    z2 = z2_ref[...]
    et = embedt_ref[...]
    e2 = e2_ref[...]
    s = jax.lax.dot_general(z.astype(jnp.bfloat16), et, (((1,), (0,)), ((), ())),
                            preferred_element_type=jnp.float32)
    dist = (z2 - 2.0 * s) + e2
    col = jax.lax.broadcasted_iota(jnp.int32, (dist.shape[0], _CHUNK), 1)
    v0, i0 = _chunk_min(dist[:, :_CHUNK], col)
    v1, i1 = _chunk_min(dist[:, _CHUNK:], col)
    v0b = v0.astype(jnp.bfloat16).astype(jnp.float32)
    out_ref[...] = jnp.where(v1 < v0b, i1 + _CHUNK, i0)


def _tc_argmin(flat, z2, embed_t, e2, n):
    return pl.pallas_call(
        _argmin_kernel,
        grid=(n // _ROWS,),
        in_specs=[
            pl.BlockSpec((_ROWS, _D), lambda t: (t, 0)),
            pl.BlockSpec((_ROWS, 1), lambda t: (t, 0)),
            pl.BlockSpec((_D, _K), lambda t: (0, 0)),
            pl.BlockSpec((1, _K), lambda t: (0, 0)),
        ],
        out_specs=pl.BlockSpec((_ROWS, 1), lambda t: (t, 0)),
        out_shape=jax.ShapeDtypeStruct((n, 1), jnp.int32),
    )(flat, z2, embed_t, e2)


_GV = 128


def _sc_gather(embed_pad, idx, n):
    mesh = plsc.VectorSubcoreMesh(core_axis_name="core",
                                  subcore_axis_name="subcore")

    @pl.kernel(out_type=jax.ShapeDtypeStruct((n, _GV), jnp.float32),
               mesh=mesh)
    def gather_kernel(x_hbm, i_hbm, o_hbm):
        def body(i_vmem, o_vmem):
            pltpu.sync_copy(x_hbm.at[i_vmem.at[0]], o_vmem)

        pltpu.emit_pipeline(
            body,
            grid=(n // _GW,),
            in_specs=[pl.BlockSpec((1, _GW), index_map=lambda i: (0, i))],
            out_specs=[pl.BlockSpec((_GW, _GV), index_map=lambda i: (i, 0))],
            core_axis_name=("core", "subcore"),
            dimension_semantics=(pltpu.PARALLEL,),
        )(i_hbm, o_hbm)

    return gather_kernel(embed_pad, idx)


def kernel(z_e, embed0, embed1):
    B, C, H, W = z_e.shape
    d = C // _NCB
    n = B * H * W
    flat = jnp.transpose(z_e.reshape(B, _NCB, d, H, W),
                         (1, 0, 3, 4, 2)).reshape(_NCB, n, d)
    z2 = jnp.sum(flat * flat, axis=2)[:, :, None]
    embeds = [embed0, embed1]
    quants = []
    for c in range(_NCB):
        e = embeds[c]
        et = e.T.astype(jnp.bfloat16)
        e2 = jnp.sum(e * e, axis=1)[None, :]
        idx = _tc_argmin(flat[c], z2[c], et, e2, n)
        e_pad = jnp.pad(e, ((0, 0), (0, _GV - d)))
        q = _sc_gather(e_pad, idx.reshape(1, n), n)
        quants.append(q[:, :d])

    quant = jnp.stack(quants)
    z_q = jnp.transpose(quant.reshape(_NCB, B, H, W, d),
                        (1, 0, 4, 2, 3)).reshape(B, C, H, W)
    return z_q


# speedup vs baseline: 1.3072x; 1.0208x over previous
import jax
import jax.numpy as jnp
from jax.experimental import pallas as pl
from jax.experimental.pallas import tpu as pltpu
from jax.experimental.pallas import tpu_sc as plsc

_ROWS = 1024
_K = 4096
_CHUNK = 2048
_D = 32
_NCB = 2
_GW = 128


def _chunk_min(dist, col):
    v = jnp.min(dist, axis=1, keepdims=True)
    i = jnp.min(jnp.where(dist <= v, col, _K), axis=1, keepdims=True)
    return v, i


def _argmin_kernel(flat_ref, z2_ref, embedt_ref, e2_ref, out_ref):
    z = flat_ref[...]
    z2 = z2_ref[...]
    et = embedt_ref[...]
    e2 = e2_ref[...]
    s = jax.lax.dot_general(z.astype(jnp.bfloat16), et, (((1,), (0,)), ((), ())),
                            preferred_element_type=jnp.float32)
    dist = (z2 - 2.0 * s) + e2
    col = jax.lax.broadcasted_iota(jnp.int32, (dist.shape[0], _CHUNK), 1)
    v0, i0 = _chunk_min(dist[:, :_CHUNK], col)
    v1, i1 = _chunk_min(dist[:, _CHUNK:], col)
    v0b = v0.astype(jnp.bfloat16).astype(jnp.float32)
    out_ref[...] = jnp.where(v1 < v0b, i1 + _CHUNK, i0)


def _tc_argmin(flat, z2, embed_t, e2, n):
    return pl.pallas_call(
        _argmin_kernel,
        grid=(n // _ROWS,),
        in_specs=[
            pl.BlockSpec((_ROWS, _D), lambda t: (t, 0)),
            pl.BlockSpec((_ROWS, 1), lambda t: (t, 0)),
            pl.BlockSpec((_D, _K), lambda t: (0, 0)),
            pl.BlockSpec((1, _K), lambda t: (0, 0)),
        ],
        out_specs=pl.BlockSpec((_ROWS, 1), lambda t: (t, 0)),
        out_shape=jax.ShapeDtypeStruct((n, 1), jnp.int32),
    )(flat, z2, embed_t, e2)


_GV = 128


def _sc_gather(embed_pad, idx, n):
    mesh = plsc.VectorSubcoreMesh(core_axis_name="core",
                                  subcore_axis_name="subcore")

    @pl.kernel(out_type=jax.ShapeDtypeStruct((n, _GV), jnp.float32),
               mesh=mesh)
    def gather_kernel(x_hbm, i_hbm, o_hbm):
        def body(i_vmem, o_vmem):
            pltpu.sync_copy(x_hbm.at[i_vmem.at[0]], o_vmem)

        pltpu.emit_pipeline(
            body,
            grid=(n // _GW,),
            in_specs=[pl.BlockSpec((1, _GW), index_map=lambda i: (0, i))],
            out_specs=[pl.BlockSpec((_GW, _GV), index_map=lambda i: (i, 0))],
            core_axis_name=("core", "subcore"),
            dimension_semantics=(pltpu.PARALLEL,),
        )(i_hbm, o_hbm)

    return gather_kernel(embed_pad, idx)


def kernel(z_e, embed0, embed1):
    B, C, H, W = z_e.shape
    d = C // _NCB
    n = B * H * W
    flat = jnp.transpose(z_e.reshape(B, _NCB, d, H, W),
                         (1, 0, 3, 4, 2)).reshape(_NCB, n, d)
    z2 = jnp.sum(flat * flat, axis=2)[:, :, None]
    embeds = [embed0, embed1]
    quants = []
    for c in range(_NCB):
        e = embeds[c]
        et = e.T.astype(jnp.bfloat16)
        e2 = jnp.sum(e * e, axis=1)[None, :]
        idx = _tc_argmin(flat[c], z2[c], et, e2, n)
        e_pad = jnp.pad(e, ((0, 0), (0, _GV - d)))
        q = _sc_gather(e_pad, idx.reshape(1, n), n)
        quants.append(q[:, :d])

    quant = jnp.stack(quants)
    z_q = jnp.transpose(quant.reshape(_NCB, B, H, W, d),
                        (1, 0, 4, 2, 3)).reshape(B, C, H, W)
    return z_q


# speedup vs baseline: 1.3292x; 1.0168x over previous
import jax
import jax.numpy as jnp
from jax.experimental import pallas as pl
from jax.experimental.pallas import tpu as pltpu
from jax.experimental.pallas import tpu_sc as plsc

_ROWS = 1024
_K = 4096
_CHUNK = 2048
_D = 32
_NCB = 2
_GW = 128


def _chunk_min(dist, hilo):
    v = jnp.min(dist, axis=1, keepdims=True)
    mask = (dist <= v).astype(jnp.bfloat16)
    st = jax.lax.dot_general(mask, hilo, (((1,), (0,)), ((), ())),
                             preferred_element_type=jnp.float32)
    i = st[:, 0:1] * 64.0 + st[:, 1:2]
    return v, i.astype(jnp.int32)


def _argmin_kernel(flat_ref, z2_ref, embedt_ref, e2_ref, hilo_ref, out_ref):
    z = flat_ref[...]
    z2 = z2_ref[...]
    et = embedt_ref[...]
    e2 = e2_ref[...]
    hilo = hilo_ref[...]
    s = jax.lax.dot_general(z.astype(jnp.bfloat16), et, (((1,), (0,)), ((), ())),
                            preferred_element_type=jnp.float32)
    dist = (z2 - 2.0 * s) + e2
    v0, i0 = _chunk_min(dist[:, :_CHUNK], hilo)
    v1, i1 = _chunk_min(dist[:, _CHUNK:], hilo)
    v0b = v0.astype(jnp.bfloat16).astype(jnp.float32)
    idx = jnp.where(v1 < v0b, i1 + _CHUNK, i0)
    out_ref[...] = jnp.minimum(idx, _K - 1)


def _tc_argmin(flat, z2, embed_t, e2, hilo, n):
    return pl.pallas_call(
        _argmin_kernel,
        grid=(n // _ROWS,),
        in_specs=[
            pl.BlockSpec((_ROWS, _D), lambda t: (t, 0)),
            pl.BlockSpec((_ROWS, 1), lambda t: (t, 0)),
            pl.BlockSpec((_D, _K), lambda t: (0, 0)),
            pl.BlockSpec((1, _K), lambda t: (0, 0)),
            pl.BlockSpec((_CHUNK, 2), lambda t: (0, 0)),
        ],
        out_specs=pl.BlockSpec((_ROWS, 1), lambda t: (t, 0)),
        out_shape=jax.ShapeDtypeStruct((n, 1), jnp.int32),
    )(flat, z2, embed_t, e2, hilo)


_GV = 128


def _sc_gather(embed_pad, idx, n):
    mesh = plsc.VectorSubcoreMesh(core_axis_name="core",
                                  subcore_axis_name="subcore")

    @pl.kernel(out_type=jax.ShapeDtypeStruct((n, _GV), jnp.float32),
               mesh=mesh)
    def gather_kernel(x_hbm, i_hbm, o_hbm):
        def body(i_vmem, o_vmem):
            pltpu.sync_copy(x_hbm.at[i_vmem.at[0]], o_vmem)

        pltpu.emit_pipeline(
            body,
            grid=(n // _GW,),
            in_specs=[pl.BlockSpec((1, _GW), index_map=lambda i: (0, i))],
            out_specs=[pl.BlockSpec((_GW, _GV), index_map=lambda i: (i, 0))],
            core_axis_name=("core", "subcore"),
            dimension_semantics=(pltpu.PARALLEL,),
        )(i_hbm, o_hbm)

    return gather_kernel(embed_pad, idx)


def kernel(z_e, embed0, embed1):
    B, C, H, W = z_e.shape
    d = C // _NCB
    n = B * H * W
    flat = jnp.transpose(z_e.reshape(B, _NCB, d, H, W),
                         (1, 0, 3, 4, 2)).reshape(_NCB, n, d)
    z2 = jnp.sum(flat * flat, axis=2)[:, :, None]
    col = jnp.arange(_CHUNK, dtype=jnp.int32)
    hilo = jnp.stack([col // 64, col % 64], axis=1).astype(jnp.bfloat16)
    embeds = [embed0, embed1]
    quants = []
    for c in range(_NCB):
        e = embeds[c]
        et = e.T.astype(jnp.bfloat16)
        e2 = jnp.sum(e * e, axis=1)[None, :]
        idx = _tc_argmin(flat[c], z2[c], et, e2, hilo, n)
        e_pad = jnp.pad(e, ((0, 0), (0, _GV - d)))
        q = _sc_gather(e_pad, idx.reshape(1, n), n)
        quants.append(q[:, :d])

    quant = jnp.stack(quants)
    z_q = jnp.transpose(quant.reshape(_NCB, B, H, W, d),
                        (1, 0, 4, 2, 3)).reshape(B, C, H, W)
    return z_q


# speedup vs baseline: 1.8979x; 1.4278x over previous
from functools import partial

import jax
import jax.numpy as jnp
from jax.experimental import pallas as pl
from jax.experimental.pallas import tpu as pltpu
from jax.experimental.pallas import tpu_sc as plsc

_K = 4096
_CHUNK = 2048
_D = 32
_NCB = 2
_GW = 128
_GV = 128


def _chunk_min(dist, hilo):
    v = jnp.min(dist, axis=0, keepdims=True)
    mask = (dist <= v).astype(jnp.bfloat16)
    st = jax.lax.dot_general(hilo, mask, (((1,), (0,)), ((), ())),
                             preferred_element_type=jnp.float32)
    i = st[0:1, :] * 64.0 + st[1:2, :]
    return v, i.astype(jnp.int32)


def _argmin_kernel(c, z_ref, embed_ref, e2_ref, hilo_ref, out_ref):
    zz = z_ref[0, 0]
    n = zz.shape[1] * zz.shape[2]
    zt = zz.reshape(_D, n)
    e = embed_ref[...]
    e2 = e2_ref[...]
    hilo = hilo_ref[...]
    z2 = jnp.sum(zt * zt, axis=0, keepdims=True)
    s = jax.lax.dot_general(e, zt.astype(jnp.bfloat16),
                            (((1,), (0,)), ((), ())),
                            preferred_element_type=jnp.float32)
    dist = (z2 - 2.0 * s) + e2
    v0, i0 = _chunk_min(dist[:_CHUNK], hilo)
    v1, i1 = _chunk_min(dist[_CHUNK:], hilo)
    v0b = v0.astype(jnp.bfloat16).astype(jnp.float32)
    idx = jnp.where(v1 < v0b, i1 + _CHUNK, i0)
    out_ref[0] = jnp.minimum(idx, _K - 1)


def _tc_argmin(z_e, embed_bf16, e2, hilo, c):
    B, C, H, W = z_e.shape
    n = H * W
    grid_kernel = partial(_argmin_kernel, c)
    out = pl.pallas_call(
        grid_kernel,
        grid=(B,),
        in_specs=[
            pl.BlockSpec((1, 1, _D, H, W), lambda t: (t, c, 0, 0, 0)),
            pl.BlockSpec((_K, _D), lambda t: (0, 0)),
            pl.BlockSpec((_K, 1), lambda t: (0, 0)),
            pl.BlockSpec((2, _CHUNK), lambda t: (0, 0)),
        ],
        out_specs=pl.BlockSpec((1, 1, n), lambda t: (t, 0, 0)),
        out_shape=jax.ShapeDtypeStruct((B, 1, n), jnp.int32),
    )(z_e.reshape(B, _NCB, _D, H, W), embed_bf16, e2, hilo)
    return out.reshape(1, B * n)


def _sc_gather(embed_pad, idx, n):
    mesh = plsc.VectorSubcoreMesh(core_axis_name="core",
                                  subcore_axis_name="subcore")

    @pl.kernel(out_type=jax.ShapeDtypeStruct((n, _GV), jnp.float32),
               mesh=mesh)
    def gather_kernel(x_hbm, i_hbm, o_hbm):
        def body(i_vmem, o_vmem):
            pltpu.sync_copy(x_hbm.at[i_vmem.at[0]], o_vmem)

        pltpu.emit_pipeline(
            body,
            grid=(n // _GW,),
            in_specs=[pl.BlockSpec((1, _GW), index_map=lambda i: (0, i))],
            out_specs=[pl.BlockSpec((_GW, _GV), index_map=lambda i: (i, 0))],
            core_axis_name=("core", "subcore"),
            dimension_semantics=(pltpu.PARALLEL,),
        )(i_hbm, o_hbm)

    return gather_kernel(embed_pad, idx)


def kernel(z_e, embed0, embed1):
    B, C, H, W = z_e.shape
    d = C // _NCB
    n = B * H * W
    col = jnp.arange(_CHUNK, dtype=jnp.int32)
    hilo = jnp.stack([col // 64, col % 64], axis=0).astype(jnp.bfloat16)
    embeds = [embed0, embed1]
    quants = []
    for c in range(_NCB):
        e = embeds[c]
        eb = e.astype(jnp.bfloat16)
        e2 = jnp.sum(e * e, axis=1)[:, None]
        idx = _tc_argmin(z_e, eb, e2, hilo, c)
        e_pad = jnp.pad(e, ((0, 0), (0, _GV - d)))
        q = _sc_gather(e_pad, idx, n)
        quants.append(q[:, :d])

    quant = jnp.stack(quants)
    z_q = jnp.transpose(quant.reshape(_NCB, B, H, W, d),
                        (1, 0, 4, 2, 3)).reshape(B, C, H, W)
    return z_q


# speedup vs baseline: 1.9093x; 1.0060x over previous
from functools import partial

import jax
import jax.numpy as jnp
from jax.experimental import pallas as pl
from jax.experimental.pallas import tpu as pltpu
from jax.experimental.pallas import tpu_sc as plsc

_K = 4096
_CHUNK = 2048
_D = 32
_NCB = 2
_GW = 256
_GV = 128


def _chunk_min(dist, hilo):
    v = jnp.min(dist, axis=0, keepdims=True)
    mask = (dist <= v).astype(jnp.bfloat16)
    st = jax.lax.dot_general(hilo, mask, (((1,), (0,)), ((), ())),
                             preferred_element_type=jnp.float32)
    i = st[0:1, :] * 64.0 + st[1:2, :]
    return v, i.astype(jnp.int32)


def _argmin_kernel(c, z_ref, embed_ref, e2_ref, hilo_ref, out_ref):
    zz = z_ref[0, 0]
    n = zz.shape[1] * zz.shape[2]
    zt = zz.reshape(_D, n)
    e = embed_ref[...]
    e2 = e2_ref[...]
    hilo = hilo_ref[...]
    z2 = jnp.sum(zt * zt, axis=0, keepdims=True)
    s = jax.lax.dot_general(e, zt.astype(jnp.bfloat16),
                            (((1,), (0,)), ((), ())),
                            preferred_element_type=jnp.float32)
    dist = (z2 - 2.0 * s) + e2
    v0, i0 = _chunk_min(dist[:_CHUNK], hilo)
    v1, i1 = _chunk_min(dist[_CHUNK:], hilo)
    v0b = v0.astype(jnp.bfloat16).astype(jnp.float32)
    idx = jnp.where(v1 < v0b, i1 + _CHUNK, i0)
    out_ref[0] = jnp.minimum(idx, _K - 1)


def _tc_argmin(z_e, embed_bf16, e2, hilo, c):
    B, C, H, W = z_e.shape
    n = H * W
    grid_kernel = partial(_argmin_kernel, c)
    out = pl.pallas_call(
        grid_kernel,
        grid=(B,),
        in_specs=[
            pl.BlockSpec((1, 1, _D, H, W), lambda t: (t, c, 0, 0, 0)),
            pl.BlockSpec((_K, _D), lambda t: (0, 0)),
            pl.BlockSpec((_K, 1), lambda t: (0, 0)),
            pl.BlockSpec((2, _CHUNK), lambda t: (0, 0)),
        ],
        out_specs=pl.BlockSpec((1, 1, n), lambda t: (t, 0, 0)),
        out_shape=jax.ShapeDtypeStruct((B, 1, n), jnp.int32),
    )(z_e.reshape(B, _NCB, _D, H, W), embed_bf16, e2, hilo)
    return out.reshape(1, B * n)


def _sc_gather(embed_pad, idx, n):
    mesh = plsc.VectorSubcoreMesh(core_axis_name="core",
                                  subcore_axis_name="subcore")

    @pl.kernel(out_type=jax.ShapeDtypeStruct((n, _GV), jnp.float32),
               mesh=mesh)
    def gather_kernel(x_hbm, i_hbm, o_hbm):
        def body(i_vmem, o_vmem):
            pltpu.sync_copy(x_hbm.at[i_vmem.at[0]], o_vmem)

        pltpu.emit_pipeline(
            body,
            grid=(n // _GW,),
            in_specs=[pl.BlockSpec((1, _GW), index_map=lambda i: (0, i))],
            out_specs=[pl.BlockSpec((_GW, _GV), index_map=lambda i: (i, 0))],
            core_axis_name=("core", "subcore"),
            dimension_semantics=(pltpu.PARALLEL,),
        )(i_hbm, o_hbm)

    return gather_kernel(embed_pad, idx)


def kernel(z_e, embed0, embed1):
    B, C, H, W = z_e.shape
    d = C // _NCB
    n = B * H * W
    col = jnp.arange(_CHUNK, dtype=jnp.int32)
    hilo = jnp.stack([col // 64, col % 64], axis=0).astype(jnp.bfloat16)
    embeds = [embed0, embed1]
    quants = []
    for c in range(_NCB):
        e = embeds[c]
        eb = e.astype(jnp.bfloat16)
        e2 = jnp.sum(e * e, axis=1)[:, None]
        idx = _tc_argmin(z_e, eb, e2, hilo, c)
        e_pad = jnp.pad(e, ((0, 0), (0, _GV - d)))
        q = _sc_gather(e_pad, idx, n)
        quants.append(q[:, :d])

    quant = jnp.stack(quants)
    z_q = jnp.transpose(quant.reshape(_NCB, B, H, W, d),
                        (1, 0, 4, 2, 3)).reshape(B, C, H, W)
    return z_q
